# baseline (speedup 1.0000x reference)
import functools
import jax
import jax.numpy as jnp
from jax import lax
from jax.experimental import pallas as pl
from jax.experimental.pallas import tpu as pltpu
from jax.experimental.pallas import tpu_sc as plsc

VOCAB_N = 1000000
EMBED_D = 64
BATCH_B = 16384
PAIR_W = 2 * EMBED_D

CHUNK = 128
LANES = 16


def _make_lookup():
    info = plsc.get_sparse_core_info()
    nw = info.num_cores * info.num_subcores
    b_per_w = BATCH_B // nw
    n_chunks = b_per_w // CHUNK
    mesh = plsc.VectorSubcoreMesh(core_axis_name="c", subcore_axis_name="s")

    @functools.partial(
        pl.kernel,
        mesh=mesh,
        out_type=jax.ShapeDtypeStruct((EMBED_D, BATCH_B), jnp.float32),
        scratch_types=[
            pltpu.VMEM((b_per_w,), jnp.int32),
            pltpu.VMEM((b_per_w,), jnp.int32),
            pltpu.VMEM((b_per_w,), jnp.int32),
            pltpu.VMEM((CHUNK, PAIR_W), jnp.float32),
            pltpu.VMEM((EMBED_D, CHUNK), jnp.float32),
            pltpu.SemaphoreType.DMA,
        ],
        compiler_params=pltpu.CompilerParams(needs_layout_passes=False),
    )
    def lookup(p_hbm, idx_hbm, outT_hbm, idx_v, pair_v, half_v, staged_v,
               blockT_v, sem):
        wid = lax.axis_index("s") * info.num_cores + lax.axis_index("c")
        base = wid * b_per_w
        pltpu.sync_copy(idx_hbm.at[pl.ds(base, b_per_w)], idx_v)

        for g in range(b_per_w // LANES):
            v = idx_v[pl.ds(g * LANES, LANES)]
            pair_v[pl.ds(g * LANES, LANES)] = lax.shift_right_logical(v, 1)
            half_v[pl.ds(g * LANES, LANES)] = lax.shift_left(
                lax.bitwise_and(v, 1), 6)

        iota = lax.iota(jnp.int32, LANES)

        def chunk_body(c, _):
            pltpu.async_copy(
                p_hbm.at[pair_v.at[pl.ds(c * CHUNK, CHUNK)]], staged_v, sem
            ).wait()
            for q in range(CHUNK // LANES):
                r_rel = iota + (q * LANES)
                h_vec = half_v[pl.ds(c * CHUNK + q * LANES, LANES)]
                for j in range(EMBED_D):
                    j_vec = jnp.full((LANES,), j, jnp.int32)
                    vals = plsc.load_gather(staged_v, [r_rel, h_vec + j_vec])
                    plsc.store_scatter(blockT_v, [j_vec, r_rel], vals)
            pltpu.sync_copy(blockT_v,
                            outT_hbm.at[:, pl.ds(base + c * CHUNK, CHUNK)])
            return _

        lax.fori_loop(0, n_chunks, chunk_body, None)

    return lookup


_lookup = _make_lookup()


def kernel(inputs, embeddings):
    paired = embeddings.reshape(VOCAB_N // 2, PAIR_W)
    outT = _lookup(paired, inputs)
    return outT.T


# speedup vs baseline: 2.0312x; 2.0312x over previous
import functools
import jax
import jax.numpy as jnp
from jax import lax
from jax.experimental import pallas as pl
from jax.experimental.pallas import tpu as pltpu
from jax.experimental.pallas import tpu_sc as plsc

VOCAB_N = 1000000
EMBED_D = 64
BATCH_B = 16384
PAIR_W = 2 * EMBED_D

CHUNK = 128
LANES = 16

_LW_BITS = 12
_LW = 1 << _LW_BITS
_HW = _LW // 2
_GRID = (VOCAB_N + _LW - 1) // _LW
P_ROWS = _GRID * _HW


def _make_lookup():
    info = plsc.get_sparse_core_info()
    nw = info.num_cores * info.num_subcores
    b_per_w = BATCH_B // nw
    n_chunks = b_per_w // CHUNK
    mesh = plsc.VectorSubcoreMesh(core_axis_name="c", subcore_axis_name="s")

    @functools.partial(
        pl.kernel,
        mesh=mesh,
        out_type=jax.ShapeDtypeStruct((EMBED_D, BATCH_B), jnp.float32),
        scratch_types=[
            pltpu.VMEM((b_per_w,), jnp.int32),
            pltpu.VMEM((b_per_w,), jnp.int32),
            pltpu.VMEM((b_per_w,), jnp.int32),
            pltpu.VMEM((CHUNK, PAIR_W), jnp.float32),
            pltpu.VMEM((EMBED_D, CHUNK), jnp.float32),
            pltpu.SemaphoreType.DMA,
        ],
        compiler_params=pltpu.CompilerParams(needs_layout_passes=False),
    )
    def lookup(p_hbm, idx_hbm, outT_hbm, idx_v, pair_v, half_v, staged_v,
               blockT_v, sem):
        wid = lax.axis_index("s") * info.num_cores + lax.axis_index("c")
        base = wid * b_per_w
        pltpu.sync_copy(idx_hbm.at[pl.ds(base, b_per_w)], idx_v)

        for g in range(b_per_w // LANES):
            v = idx_v[pl.ds(g * LANES, LANES)]
            pair_v[pl.ds(g * LANES, LANES)] = lax.bitwise_or(
                lax.shift_left(lax.shift_right_logical(v, _LW_BITS),
                               _LW_BITS - 1),
                lax.bitwise_and(v, _HW - 1))
            half_v[pl.ds(g * LANES, LANES)] = lax.shift_left(
                lax.bitwise_and(lax.shift_right_logical(v, _LW_BITS - 1), 1),
                6)

        iota = lax.iota(jnp.int32, LANES)

        def chunk_body(c, _):
            pltpu.async_copy(
                p_hbm.at[pair_v.at[pl.ds(c * CHUNK, CHUNK)]], staged_v, sem
            ).wait()
            for q in range(CHUNK // LANES):
                r_rel = iota + (q * LANES)
                h_vec = half_v[pl.ds(c * CHUNK + q * LANES, LANES)]
                for j in range(EMBED_D):
                    j_vec = jnp.full((LANES,), j, jnp.int32)
                    vals = plsc.load_gather(staged_v, [r_rel, h_vec + j_vec])
                    plsc.store_scatter(blockT_v, [j_vec, r_rel], vals)
            pltpu.sync_copy(blockT_v,
                            outT_hbm.at[:, pl.ds(base + c * CHUNK, CHUNK)])
            return _

        lax.fori_loop(0, n_chunks, chunk_body, None)

    return lookup


_lookup = _make_lookup()


def _repack_body(x_ref, o_ref):
    x = x_ref[...]
    xx = jnp.concatenate([x[:, :_HW], x[:, _HW:]], axis=0)
    o_ref[...] = lax.transpose(xx, (1, 0))


def _repack(embT):
    return pl.pallas_call(
        _repack_body,
        grid=(_GRID,),
        in_specs=[pl.BlockSpec((EMBED_D, _LW), lambda w: (0, w))],
        out_specs=pl.BlockSpec((_HW, PAIR_W), lambda w: (w, 0)),
        out_shape=jax.ShapeDtypeStruct((P_ROWS, PAIR_W), jnp.float32),
    )(embT)


def kernel(inputs, embeddings):
    paired = _repack(embeddings.T)
    outT = _lookup(paired, inputs)
    return outT.T


# speedup vs baseline: 2.6656x; 1.3123x over previous
import functools
import jax
import jax.numpy as jnp
from jax import lax
from jax.experimental import pallas as pl
from jax.experimental.pallas import tpu as pltpu
from jax.experimental.pallas import tpu_sc as plsc

VOCAB_N = 1000000
EMBED_D = 64
BATCH_B = 16384
PAIR_W = 2 * EMBED_D

CHUNK = 128
LANES = 16

_LW_BITS = 13
_LW = 1 << _LW_BITS
_HW = _LW // 2
_GRID = (VOCAB_N + _LW - 1) // _LW
P_ROWS = _GRID * _HW


def _make_lookup():
    info = plsc.get_sparse_core_info()
    nw = info.num_cores * info.num_subcores
    b_per_w = BATCH_B // nw
    n_chunks = b_per_w // CHUNK
    mesh = plsc.VectorSubcoreMesh(core_axis_name="c", subcore_axis_name="s")

    @functools.partial(
        pl.kernel,
        mesh=mesh,
        out_type=jax.ShapeDtypeStruct((EMBED_D, BATCH_B), jnp.float32),
        scratch_types=[
            pltpu.VMEM((b_per_w,), jnp.int32),
            pltpu.VMEM((b_per_w,), jnp.int32),
            pltpu.VMEM((b_per_w,), jnp.int32),
            pltpu.VMEM((CHUNK, PAIR_W), jnp.float32),
            pltpu.VMEM((EMBED_D, CHUNK), jnp.float32),
            pltpu.SemaphoreType.DMA,
        ],
        compiler_params=pltpu.CompilerParams(needs_layout_passes=False),
    )
    def lookup(p_hbm, idx_hbm, outT_hbm, idx_v, pair_v, half_v, staged_v,
               blockT_v, sem):
        wid = lax.axis_index("s") * info.num_cores + lax.axis_index("c")
        base = wid * b_per_w
        pltpu.sync_copy(idx_hbm.at[pl.ds(base, b_per_w)], idx_v)

        for g in range(b_per_w // LANES):
            v = idx_v[pl.ds(g * LANES, LANES)]
            pair_v[pl.ds(g * LANES, LANES)] = lax.bitwise_or(
                lax.shift_left(lax.shift_right_logical(v, _LW_BITS),
                               _LW_BITS - 1),
                lax.bitwise_and(v, _HW - 1))
            half_v[pl.ds(g * LANES, LANES)] = lax.shift_left(
                lax.bitwise_and(lax.shift_right_logical(v, _LW_BITS - 1), 1),
                6)

        iota = lax.iota(jnp.int32, LANES)

        def chunk_body(c, _):
            pltpu.async_copy(
                p_hbm.at[pair_v.at[pl.ds(c * CHUNK, CHUNK)]], staged_v, sem
            ).wait()
            for q in range(CHUNK // LANES):
                r_rel = iota + (q * LANES)
                h_vec = half_v[pl.ds(c * CHUNK + q * LANES, LANES)]
                for j in range(EMBED_D):
                    j_vec = jnp.full((LANES,), j, jnp.int32)
                    vals = plsc.load_gather(staged_v, [r_rel, h_vec + j_vec])
                    plsc.store_scatter(blockT_v, [j_vec, r_rel], vals)
            pltpu.sync_copy(blockT_v,
                            outT_hbm.at[:, pl.ds(base + c * CHUNK, CHUNK)])
            return _

        lax.fori_loop(0, n_chunks, chunk_body, None)

    return lookup


_lookup = _make_lookup()


def _repack_body(x_ref, o_ref):
    x = x_ref[...]
    xx = jnp.concatenate([x[:, :_HW], x[:, _HW:]], axis=0)
    o_ref[...] = lax.transpose(xx, (1, 0))


def _repack(embT):
    return pl.pallas_call(
        _repack_body,
        grid=(_GRID,),
        in_specs=[pl.BlockSpec((EMBED_D, _LW), lambda w: (0, w))],
        out_specs=pl.BlockSpec((_HW, PAIR_W), lambda w: (w, 0)),
        out_shape=jax.ShapeDtypeStruct((P_ROWS, PAIR_W), jnp.float32),
    )(embT)


def kernel(inputs, embeddings):
    paired = _repack(embeddings.T)
    outT = _lookup(paired, inputs)
    return outT.T


# speedup vs baseline: 3.0081x; 1.1285x over previous
import functools
import jax
import jax.numpy as jnp
from jax import lax
from jax.experimental import pallas as pl
from jax.experimental.pallas import tpu as pltpu
from jax.experimental.pallas import tpu_sc as plsc

VOCAB_N = 1000000
EMBED_D = 64
BATCH_B = 16384
PAIR_W = 2 * EMBED_D

CHUNK = 128
LANES = 16

_LW_BITS = 14
_LW = 1 << _LW_BITS
_HW = _LW // 2
_GRID = (VOCAB_N + _LW - 1) // _LW
P_ROWS = _GRID * _HW


def _make_lookup():
    info = plsc.get_sparse_core_info()
    nw = info.num_cores * info.num_subcores
    b_per_w = BATCH_B // nw
    n_chunks = b_per_w // CHUNK
    mesh = plsc.VectorSubcoreMesh(core_axis_name="c", subcore_axis_name="s")

    @functools.partial(
        pl.kernel,
        mesh=mesh,
        out_type=jax.ShapeDtypeStruct((EMBED_D, BATCH_B), jnp.float32),
        scratch_types=[
            pltpu.VMEM((b_per_w,), jnp.int32),
            pltpu.VMEM((b_per_w,), jnp.int32),
            pltpu.VMEM((b_per_w,), jnp.int32),
            pltpu.VMEM((CHUNK, PAIR_W), jnp.float32),
            pltpu.VMEM((EMBED_D, CHUNK), jnp.float32),
            pltpu.SemaphoreType.DMA,
        ],
        compiler_params=pltpu.CompilerParams(needs_layout_passes=False),
    )
    def lookup(p_hbm, idx_hbm, outT_hbm, idx_v, pair_v, half_v, staged_v,
               blockT_v, sem):
        wid = lax.axis_index("s") * info.num_cores + lax.axis_index("c")
        base = wid * b_per_w
        pltpu.sync_copy(idx_hbm.at[pl.ds(base, b_per_w)], idx_v)

        for g in range(b_per_w // LANES):
            v = idx_v[pl.ds(g * LANES, LANES)]
            pair_v[pl.ds(g * LANES, LANES)] = lax.bitwise_or(
                lax.shift_left(lax.shift_right_logical(v, _LW_BITS),
                               _LW_BITS - 1),
                lax.bitwise_and(v, _HW - 1))
            half_v[pl.ds(g * LANES, LANES)] = lax.shift_left(
                lax.bitwise_and(lax.shift_right_logical(v, _LW_BITS - 1), 1),
                6)

        iota = lax.iota(jnp.int32, LANES)

        def chunk_body(c, _):
            pltpu.async_copy(
                p_hbm.at[pair_v.at[pl.ds(c * CHUNK, CHUNK)]], staged_v, sem
            ).wait()
            for q in range(CHUNK // LANES):
                r_rel = iota + (q * LANES)
                h_vec = half_v[pl.ds(c * CHUNK + q * LANES, LANES)]
                for j in range(EMBED_D):
                    j_vec = jnp.full((LANES,), j, jnp.int32)
                    vals = plsc.load_gather(staged_v, [r_rel, h_vec + j_vec])
                    plsc.store_scatter(blockT_v, [j_vec, r_rel], vals)
            pltpu.sync_copy(blockT_v,
                            outT_hbm.at[:, pl.ds(base + c * CHUNK, CHUNK)])
            return _

        lax.fori_loop(0, n_chunks, chunk_body, None)

    return lookup


_lookup = _make_lookup()


def _repack_body(x_ref, o_ref):
    x = x_ref[...]
    xx = jnp.concatenate([x[:, :_HW], x[:, _HW:]], axis=0)
    o_ref[...] = lax.transpose(xx, (1, 0))


def _repack(embT):
    return pl.pallas_call(
        _repack_body,
        grid=(_GRID,),
        in_specs=[pl.BlockSpec((EMBED_D, _LW), lambda w: (0, w))],
        out_specs=pl.BlockSpec((_HW, PAIR_W), lambda w: (w, 0)),
        out_shape=jax.ShapeDtypeStruct((P_ROWS, PAIR_W), jnp.float32),
    )(embT)


def kernel(inputs, embeddings):
    paired = _repack(embeddings.T)
    outT = _lookup(paired, inputs)
    return outT.T


# speedup vs baseline: 3.0736x; 1.0218x over previous
import functools
import jax
import jax.numpy as jnp
from jax import lax
from jax.experimental import pallas as pl
from jax.experimental.pallas import tpu as pltpu
from jax.experimental.pallas import tpu_sc as plsc

VOCAB_N = 1000000
EMBED_D = 64
BATCH_B = 16384
PAIR_W = 2 * EMBED_D

CHUNK = 128
LANES = 16

_LW_BITS = 15
_LW = 1 << _LW_BITS
_HW = _LW // 2
_GRID = (VOCAB_N + _LW - 1) // _LW
P_ROWS = _GRID * _HW


def _make_lookup():
    info = plsc.get_sparse_core_info()
    nw = info.num_cores * info.num_subcores
    b_per_w = BATCH_B // nw
    n_chunks = b_per_w // CHUNK
    mesh = plsc.VectorSubcoreMesh(core_axis_name="c", subcore_axis_name="s")

    @functools.partial(
        pl.kernel,
        mesh=mesh,
        out_type=jax.ShapeDtypeStruct((EMBED_D, BATCH_B), jnp.float32),
        scratch_types=[
            pltpu.VMEM((b_per_w,), jnp.int32),
            pltpu.VMEM((b_per_w,), jnp.int32),
            pltpu.VMEM((b_per_w,), jnp.int32),
            pltpu.VMEM((CHUNK, PAIR_W), jnp.float32),
            pltpu.VMEM((EMBED_D, CHUNK), jnp.float32),
            pltpu.SemaphoreType.DMA,
        ],
        compiler_params=pltpu.CompilerParams(needs_layout_passes=False),
    )
    def lookup(p_hbm, idx_hbm, outT_hbm, idx_v, pair_v, half_v, staged_v,
               blockT_v, sem):
        wid = lax.axis_index("s") * info.num_cores + lax.axis_index("c")
        base = wid * b_per_w
        pltpu.sync_copy(idx_hbm.at[pl.ds(base, b_per_w)], idx_v)

        for g in range(b_per_w // LANES):
            v = idx_v[pl.ds(g * LANES, LANES)]
            pair_v[pl.ds(g * LANES, LANES)] = lax.bitwise_or(
                lax.shift_left(lax.shift_right_logical(v, _LW_BITS),
                               _LW_BITS - 1),
                lax.bitwise_and(v, _HW - 1))
            half_v[pl.ds(g * LANES, LANES)] = lax.shift_left(
                lax.bitwise_and(lax.shift_right_logical(v, _LW_BITS - 1), 1),
                6)

        iota = lax.iota(jnp.int32, LANES)

        def chunk_body(c, _):
            pltpu.async_copy(
                p_hbm.at[pair_v.at[pl.ds(c * CHUNK, CHUNK)]], staged_v, sem
            ).wait()
            for q in range(CHUNK // LANES):
                r_rel = iota + (q * LANES)
                h_vec = half_v[pl.ds(c * CHUNK + q * LANES, LANES)]
                for j in range(EMBED_D):
                    j_vec = jnp.full((LANES,), j, jnp.int32)
                    vals = plsc.load_gather(staged_v, [r_rel, h_vec + j_vec])
                    plsc.store_scatter(blockT_v, [j_vec, r_rel], vals)
            pltpu.sync_copy(blockT_v,
                            outT_hbm.at[:, pl.ds(base + c * CHUNK, CHUNK)])
            return _

        lax.fori_loop(0, n_chunks, chunk_body, None)

    return lookup


_lookup = _make_lookup()


def _repack_body(x_ref, o_ref):
    x = x_ref[...]
    xx = jnp.concatenate([x[:, :_HW], x[:, _HW:]], axis=0)
    o_ref[...] = lax.transpose(xx, (1, 0))


def _repack(embT):
    return pl.pallas_call(
        _repack_body,
        grid=(_GRID,),
        in_specs=[pl.BlockSpec((EMBED_D, _LW), lambda w: (0, w))],
        out_specs=pl.BlockSpec((_HW, PAIR_W), lambda w: (w, 0)),
        out_shape=jax.ShapeDtypeStruct((P_ROWS, PAIR_W), jnp.float32),
    )(embT)


def kernel(inputs, embeddings):
    paired = _repack(embeddings.T)
    outT = _lookup(paired, inputs)
    return outT.T


# speedup vs baseline: 3.0989x; 1.0082x over previous
import functools
import jax
import jax.numpy as jnp
from jax import lax
from jax.experimental import pallas as pl
from jax.experimental.pallas import tpu as pltpu
from jax.experimental.pallas import tpu_sc as plsc

VOCAB_N = 1000000
EMBED_D = 64
BATCH_B = 16384
PAIR_W = 2 * EMBED_D

CHUNK = 128
LANES = 16

_LW_BITS = 15
_LW = 1 << _LW_BITS
_HW = _LW // 2
_GRID = (VOCAB_N + _LW - 1) // _LW
P_ROWS = _GRID * _HW


def _make_lookup():
    info = plsc.get_sparse_core_info()
    nw = info.num_cores * info.num_subcores
    b_per_w = BATCH_B // nw
    n_chunks = b_per_w // CHUNK
    mesh = plsc.VectorSubcoreMesh(core_axis_name="c", subcore_axis_name="s")

    @functools.partial(
        pl.kernel,
        mesh=mesh,
        out_type=jax.ShapeDtypeStruct((EMBED_D, BATCH_B), jnp.float32),
        scratch_types=[
            pltpu.VMEM((b_per_w,), jnp.int32),
            pltpu.VMEM((b_per_w,), jnp.int32),
            pltpu.VMEM((b_per_w,), jnp.int32),
            pltpu.VMEM((b_per_w, PAIR_W), jnp.float32),
            pltpu.VMEM((EMBED_D, b_per_w), jnp.float32),
            pltpu.SemaphoreType.DMA,
        ],
        compiler_params=pltpu.CompilerParams(needs_layout_passes=False),
    )
    def lookup(p_hbm, idx_hbm, outT_hbm, idx_v, pair_v, half_v, staged_v,
               blockT_v, sem):
        wid = lax.axis_index("s") * info.num_cores + lax.axis_index("c")
        base = wid * b_per_w
        pltpu.sync_copy(idx_hbm.at[pl.ds(base, b_per_w)], idx_v)

        for g in range(b_per_w // LANES):
            v = idx_v[pl.ds(g * LANES, LANES)]
            pair_v[pl.ds(g * LANES, LANES)] = lax.bitwise_or(
                lax.shift_left(lax.shift_right_logical(v, _LW_BITS),
                               _LW_BITS - 1),
                lax.bitwise_and(v, _HW - 1))
            half_v[pl.ds(g * LANES, LANES)] = lax.shift_left(
                lax.bitwise_and(lax.shift_right_logical(v, _LW_BITS - 1), 1),
                6)

        iota = lax.iota(jnp.int32, LANES)

        pltpu.async_copy(p_hbm.at[pair_v], staged_v, sem).wait()

        def extract_body(g, _):
            for q in range(CHUNK // LANES):
                r_rel = iota + (g * CHUNK + q * LANES)
                h_vec = half_v[pl.ds(g * CHUNK + q * LANES, LANES)]
                for j in range(EMBED_D):
                    j_vec = jnp.full((LANES,), j, jnp.int32)
                    vals = plsc.load_gather(staged_v, [r_rel, h_vec + j_vec])
                    plsc.store_scatter(blockT_v, [j_vec, r_rel], vals)
            return _

        lax.fori_loop(0, n_chunks, extract_body, None)
        pltpu.sync_copy(blockT_v, outT_hbm.at[:, pl.ds(base, b_per_w)])

    return lookup


_lookup = _make_lookup()


def _repack_body(x_ref, o_ref):
    x = x_ref[...]
    xx = jnp.concatenate([x[:, :_HW], x[:, _HW:]], axis=0)
    o_ref[...] = lax.transpose(xx, (1, 0))


def _repack(embT):
    return pl.pallas_call(
        _repack_body,
        grid=(_GRID,),
        in_specs=[pl.BlockSpec((EMBED_D, _LW), lambda w: (0, w))],
        out_specs=pl.BlockSpec((_HW, PAIR_W), lambda w: (w, 0)),
        out_shape=jax.ShapeDtypeStruct((P_ROWS, PAIR_W), jnp.float32),
    )(embT)


def kernel(inputs, embeddings):
    paired = _repack(embeddings.T)
    outT = _lookup(paired, inputs)
    return outT.T
